# baseline (speedup 1.0000x reference)
import functools

import jax
import jax.numpy as jnp
from jax import lax
from jax.experimental import pallas as pl
from jax.experimental.pallas import tpu as pltpu
from jax.experimental.pallas import tpu_sc as plsc

EMB_DIM = 64
IDX_W = 128
ROWS_PER_CHUNK = 8


@functools.partial(jax.jit, static_argnames=("n_rows", "rows_per_worker"))
def _embed_lookup(x_idx, table, n_rows, rows_per_worker):
    mesh = plsc.VectorSubcoreMesh(core_axis_name="c", subcore_axis_name="s")
    info = plsc.get_sparse_core_info()
    nc = info.num_cores

    n_chunks = rows_per_worker // ROWS_PER_CHUNK

    def body(idx_hbm, table_hbm, out_hbm, idx_v, rows_v, sem):
        wid = lax.axis_index("s") * nc + lax.axis_index("c")
        base = wid * rows_per_worker

        def chunk(i, carry):
            off = base + i * ROWS_PER_CHUNK
            pltpu.sync_copy(idx_hbm.at[pl.ds(off, ROWS_PER_CHUNK)], idx_v)
            copies = [
                pltpu.async_copy(table_hbm.at[idx_v.at[j]], rows_v.at[j], sem)
                for j in range(ROWS_PER_CHUNK)
            ]
            for c in copies:
                c.wait()
            pltpu.sync_copy(rows_v, out_hbm.at[pl.ds(off, ROWS_PER_CHUNK)])
            return carry

        lax.fori_loop(0, n_chunks, chunk, 0)

    run = pl.kernel(
        body,
        out_type=jax.ShapeDtypeStruct((n_rows, IDX_W, EMB_DIM), jnp.float32),
        mesh=mesh,
        scratch_types=[
            pltpu.VMEM((ROWS_PER_CHUNK, IDX_W), jnp.int32),
            pltpu.VMEM((ROWS_PER_CHUNK, IDX_W, EMB_DIM), jnp.float32),
            pltpu.SemaphoreType.DMA,
        ],
        compiler_params=pltpu.CompilerParams(use_tc_tiling_on_sc=False),
    )
    return run(x_idx, table)


def kernel(x, table):
    b, l = x.shape
    total = b * l
    n_rows = total // IDX_W
    x_idx = x.reshape(n_rows, IDX_W).astype(jnp.int32)
    info = plsc.get_sparse_core_info()
    n_workers = info.num_cores * info.num_subcores
    rows_per_worker = n_rows // n_workers
    out = _embed_lookup(x_idx, table, n_rows, rows_per_worker)
    return out.reshape(b, l, EMB_DIM)


# speedup vs baseline: 1.0192x; 1.0192x over previous
import functools

import jax
import jax.numpy as jnp
from jax import lax
from jax.experimental import pallas as pl
from jax.experimental.pallas import tpu as pltpu
from jax.experimental.pallas import tpu_sc as plsc

EMB_DIM = 64
IDX_W = 128
NSLOT = 10
LEAD = 5


@functools.partial(jax.jit, static_argnames=("n_rows", "rows_per_worker"))
def _embed_lookup(x_idx, table, n_rows, rows_per_worker):
    mesh = plsc.VectorSubcoreMesh(core_axis_name="c", subcore_axis_name="s")
    info = plsc.get_sparse_core_info()
    nc = info.num_cores

    n_outer = rows_per_worker // NSLOT

    def body(idx_hbm, table_hbm, out_hbm, idx_v, rows_v, sem_g, sem_s):
        wid = lax.axis_index("s") * nc + lax.axis_index("c")
        base = wid * rows_per_worker
        pltpu.sync_copy(idx_hbm.at[pl.ds(base, rows_per_worker)], idx_v)

        def fire_gather(c, slot):
            pltpu.async_copy(table_hbm.at[idx_v.at[c]], rows_v.at[slot],
                             sem_g.at[slot])

        def wait_gather(slot):
            pltpu.make_async_copy(table_hbm.at[idx_v.at[0]], rows_v.at[slot],
                                  sem_g.at[slot]).wait()

        def fire_store(c, slot):
            pltpu.async_copy(rows_v.at[slot], out_hbm.at[base + c],
                             sem_s.at[slot])

        def wait_store(slot):
            pltpu.make_async_copy(rows_v.at[slot], out_hbm.at[base],
                                  sem_s.at[slot]).wait()

        for c in range(LEAD):
            fire_gather(c, c)

        def outer(o, carry):
            for b in range(NSLOT):
                i = o * NSLOT + b
                wait_gather(b)
                fire_store(i, b)
                ns = (b + LEAD) % NSLOT
                if b >= LEAD:
                    wait_store(ns)

                    @pl.when(o < n_outer - 1)
                    def _():
                        fire_gather(i + LEAD, ns)
                else:
                    @pl.when(o > 0)
                    def _():
                        wait_store(ns)

                    fire_gather(i + LEAD, ns)
            return carry

        lax.fori_loop(0, n_outer, outer, 0)

        for b in range(LEAD):
            wait_store((b + LEAD) % NSLOT)

    run = pl.kernel(
        body,
        out_type=jax.ShapeDtypeStruct((n_rows, IDX_W, EMB_DIM), jnp.float32),
        mesh=mesh,
        scratch_types=[
            pltpu.VMEM((rows_per_worker, IDX_W), jnp.int32),
            pltpu.VMEM((NSLOT, IDX_W, EMB_DIM), jnp.float32),
            pltpu.SemaphoreType.DMA((NSLOT,)),
            pltpu.SemaphoreType.DMA((NSLOT,)),
        ],
        compiler_params=pltpu.CompilerParams(use_tc_tiling_on_sc=False),
    )
    return run(x_idx, table)


def kernel(x, table):
    b, l = x.shape
    total = b * l
    n_rows = total // IDX_W
    x_idx = x.reshape(n_rows, IDX_W).astype(jnp.int32)
    info = plsc.get_sparse_core_info()
    n_workers = info.num_cores * info.num_subcores
    rows_per_worker = n_rows // n_workers
    out = _embed_lookup(x_idx, table, n_rows, rows_per_worker)
    return out.reshape(b, l, EMB_DIM)


# speedup vs baseline: 1.0206x; 1.0014x over previous
import functools

import jax
import jax.numpy as jnp
from jax import lax
from jax.experimental import pallas as pl
from jax.experimental.pallas import tpu as pltpu
from jax.experimental.pallas import tpu_sc as plsc

EMB_DIM = 64
IDX_W = 128
NSLOT = 10
LEAD = 5


@functools.partial(jax.jit, static_argnames=("n_idx", "idx_per_worker"))
def _embed_lookup(x_idx, table, n_idx, idx_per_worker):
    mesh = plsc.VectorSubcoreMesh(core_axis_name="c", subcore_axis_name="s")
    info = plsc.get_sparse_core_info()
    nc = info.num_cores

    n_chunks = idx_per_worker // IDX_W
    n_outer = n_chunks // NSLOT

    def body(idx_hbm, table_hbm, out_hbm, idx_v, rows_v, sem_g, sem_s):
        wid = lax.axis_index("s") * nc + lax.axis_index("c")
        base = wid * idx_per_worker
        pltpu.sync_copy(idx_hbm.at[pl.ds(base, idx_per_worker)], idx_v)

        def fire_gather(c, slot):
            pltpu.async_copy(table_hbm.at[idx_v.at[pl.ds(c * IDX_W, IDX_W)]],
                             rows_v.at[slot], sem_g.at[slot])

        def wait_gather(slot):
            pltpu.make_async_copy(table_hbm.at[idx_v.at[pl.ds(0, IDX_W)]],
                                  rows_v.at[slot], sem_g.at[slot]).wait()

        def fire_store(c, slot):
            pltpu.async_copy(rows_v.at[slot],
                             out_hbm.at[pl.ds(base + c * IDX_W, IDX_W)],
                             sem_s.at[slot])

        def wait_store(slot):
            pltpu.make_async_copy(rows_v.at[slot],
                                  out_hbm.at[pl.ds(base, IDX_W)],
                                  sem_s.at[slot]).wait()

        for c in range(LEAD):
            fire_gather(c, c)

        def outer(o, carry):
            for b in range(NSLOT):
                i = o * NSLOT + b
                wait_gather(b)
                fire_store(i, b)
                ns = (b + LEAD) % NSLOT
                if b >= LEAD:
                    wait_store(ns)

                    @pl.when(o < n_outer - 1)
                    def _():
                        fire_gather(i + LEAD, ns)
                else:
                    @pl.when(o > 0)
                    def _():
                        wait_store(ns)

                    fire_gather(i + LEAD, ns)
            return carry

        lax.fori_loop(0, n_outer, outer, 0)

        for b in range(LEAD):
            wait_store((b + LEAD) % NSLOT)

    run = pl.kernel(
        body,
        out_type=jax.ShapeDtypeStruct((n_idx, EMB_DIM), jnp.float32),
        mesh=mesh,
        scratch_types=[
            pltpu.VMEM((idx_per_worker,), jnp.int32),
            pltpu.VMEM((NSLOT, IDX_W, EMB_DIM), jnp.float32),
            pltpu.SemaphoreType.DMA((NSLOT,)),
            pltpu.SemaphoreType.DMA((NSLOT,)),
        ],
        compiler_params=pltpu.CompilerParams(use_tc_tiling_on_sc=False),
    )
    return run(x_idx, table)


def kernel(x, table):
    b, l = x.shape
    total = b * l
    x_idx = x.reshape(total).astype(jnp.int32)
    info = plsc.get_sparse_core_info()
    n_workers = info.num_cores * info.num_subcores
    idx_per_worker = total // n_workers
    out = _embed_lookup(x_idx, table, total, idx_per_worker)
    return out.reshape(b, l, EMB_DIM)
